# baseline (speedup 1.0000x reference)
import jax
import jax.numpy as jnp
from jax import lax
from jax.experimental import pallas as pl
from jax.experimental.pallas import tpu as pltpu
from jax.experimental.pallas import tpu_sc as plsc

N_NODES = 10000
N_EDGES = 320000
D = 128

NC = 2
NS = 16
CHUNK = 40
ROWS_PER_TILE = N_EDGES // (NC * NS * CHUNK)
PIECE = CHUNK
NPIECES = N_NODES // PIECE
PIECES_PER_TILE = -(-NPIECES // NS)
CN = 10240
CPIECE = 2048


NB = 3


def _sc_body(src_hbm, idx_hbm, sums_hbm, counts_hbm,
             acc, cnt, idx_v, row0, row1, row2, cstage_v, ones_v,
             ld0, ld1, ld2, sc0, sc1, sc2, ct0, ct1, ct2, idxsem):
    rows = (row0, row1, row2)
    ldsem = (ld0, ld1, ld2)
    scsem = (sc0, sc1, sc2)
    ctsem = (ct0, ct1, ct2)
    zbuf = row2
    c = lax.axis_index("c")
    s = lax.axis_index("s")

    wid = c * NS + s
    base_row = wid * ROWS_PER_TILE

    def src_slice(cur):
        return src_hbm.at[pl.ds((base_row + cur) * CHUNK, CHUNK)]

    idx_cp = pltpu.make_async_copy(idx_hbm.at[1, wid], idx_v, idxsem)
    idx_cp.start()
    pltpu.async_copy(src_slice(0), rows[0], ldsem[0])
    pltpu.async_copy(src_slice(1), rows[1], ldsem[1])

    zf = jnp.zeros((16,), jnp.float32)

    @pl.loop(0, CHUNK * D // 16)
    def _(i):
        zbuf[i // (D // 16), pl.ds((i % (D // 16)) * 16, 16)] = zf

    @pl.loop(0, CHUNK // 16)
    def _(i):
        ones_v[pl.ds(i * 16, 16)] = jnp.ones((16,), jnp.float32)

    for k in range(PIECES_PER_TILE):
        p = s + k * NS

        @pl.when(p < NPIECES)
        def _():
            pltpu.sync_copy(zbuf, acc.at[pl.ds(p * PIECE, PIECE)])

    @pl.when(s < CN // CPIECE)
    def _():
        @pl.loop(0, CPIECE // 16)
        def _(i):
            cstage_v[pl.ds(i * 16, 16)] = zf
        pltpu.sync_copy(cstage_v, cnt.at[pl.ds(s * CPIECE, CPIECE)])

    plsc.subcore_barrier()

    idx_cp.wait()


    @pl.loop(0, ROWS_PER_TILE, step=NB)
    def _(r0):
        for b in range(NB):
            cur = r0 + b

            @pl.when(cur < ROWS_PER_TILE)
            def _():
                pltpu.make_async_copy(src_slice(cur), rows[b], ldsem[b]).wait()
                idx_row = idx_v.at[cur]
                pltpu.async_copy(rows[b], acc.at[idx_row], scsem[b], add=True)
                pltpu.async_copy(ones_v, cnt.at[idx_row], ctsem[b], add=True)
                bn = (b + 2) % NB

                @pl.when(cur >= 1)
                def _():
                    prev_idx = idx_v.at[cur - 1]
                    pltpu.make_async_copy(
                        rows[bn], acc.at[prev_idx], scsem[bn]).wait()
                    pltpu.make_async_copy(
                        ones_v, cnt.at[prev_idx], ctsem[bn]).wait()

                @pl.when(cur + 2 < ROWS_PER_TILE)
                def _():
                    pltpu.async_copy(src_slice(cur + 2), rows[bn], ldsem[bn])

    last = ROWS_PER_TILE - 1
    bl = last % NB
    last_idx = idx_v.at[last]
    pltpu.make_async_copy(rows[bl], acc.at[last_idx], scsem[bl]).wait()
    pltpu.make_async_copy(ones_v, cnt.at[last_idx], ctsem[bl]).wait()

    plsc.subcore_barrier()

    for k in range(PIECES_PER_TILE):
        p = s + k * NS

        @pl.when(p < NPIECES)
        def _():
            off = p * PIECE
            pltpu.async_copy(
                acc.at[pl.ds(off, PIECE)], sums_hbm.at[c, pl.ds(off, PIECE)],
                idxsem)

    @pl.when(s < CN // CPIECE)
    def _():
        pltpu.async_copy(
            cnt.at[pl.ds(s * CPIECE, CPIECE)],
            counts_hbm.at[pl.ds(c * CN + s * CPIECE, CPIECE)], idxsem)

    for k in range(PIECES_PER_TILE):
        p = s + k * NS

        @pl.when(p < NPIECES)
        def _():
            off = p * PIECE
            pltpu.make_async_copy(
                acc.at[pl.ds(off, PIECE)], sums_hbm.at[c, pl.ds(off, PIECE)],
                idxsem).wait()

    @pl.when(s < CN // CPIECE)
    def _():
        pltpu.make_async_copy(
            cnt.at[pl.ds(s * CPIECE, CPIECE)],
            counts_hbm.at[pl.ds(c * CN + s * CPIECE, CPIECE)], idxsem).wait()


_sc_agg = pl.kernel(
    _sc_body,
    out_type=(
        jax.ShapeDtypeStruct((NC, N_NODES, D), jnp.float32),
        jax.ShapeDtypeStruct((NC * CN,), jnp.float32),
    ),
    mesh=plsc.VectorSubcoreMesh(core_axis_name="c", subcore_axis_name="s"),
    scratch_types=[
        pltpu.VMEM_SHARED((N_NODES, D), jnp.float32),
        pltpu.VMEM_SHARED((CN,), jnp.float32),
        pltpu.VMEM((ROWS_PER_TILE, CHUNK), jnp.int32),
        pltpu.VMEM((CHUNK, D), jnp.float32),
        pltpu.VMEM((CHUNK, D), jnp.float32),
        pltpu.VMEM((CHUNK, D), jnp.float32),
        pltpu.VMEM((CPIECE,), jnp.float32),
        pltpu.VMEM((CHUNK,), jnp.float32),
    ] + [pltpu.SemaphoreType.DMA] * 10,
)

ROWS_BLK = 2000


def _tc_body(sums_ref, counts_ref, w_ref, b_ref, out_ref):
    total = sums_ref[0] + sums_ref[1]
    cnt = counts_ref[:, 0:1] + counts_ref[:, 1:2]
    mean = total / jnp.maximum(cnt, 1.0)
    out_ref[...] = lax.dot_general(
        mean, w_ref[...], (((1,), (1,)), ((), ())),
        preferred_element_type=jnp.float32,
        precision=lax.Precision.HIGHEST,
    ) + b_ref[...]


_tc_linear = pl.pallas_call(
    _tc_body,
    grid=(N_NODES // ROWS_BLK,),
    in_specs=[
        pl.BlockSpec((NC, ROWS_BLK, D), lambda i: (0, i, 0)),
        pl.BlockSpec((ROWS_BLK, NC), lambda i: (i, 0)),
        pl.BlockSpec((D, D), lambda i: (0, 0)),
        pl.BlockSpec((1, D), lambda i: (0, 0)),
    ],
    out_specs=pl.BlockSpec((ROWS_BLK, D), lambda i: (i, 0)),
    out_shape=jax.ShapeDtypeStruct((N_NODES, D), jnp.float32),
)


def kernel(source_node_representation_with_coefficient, edge_index, feature_dim, W, b):
    src = source_node_representation_with_coefficient
    idx4d = edge_index.astype(jnp.int32).reshape(2, NC * NS, ROWS_PER_TILE, CHUNK)
    sums, counts_flat = _sc_agg(src, idx4d)
    counts_t = counts_flat.reshape(NC, CN)[:, :N_NODES].T
    return _tc_linear(sums, counts_t, W, b.reshape(1, D))


# speedup vs baseline: 1.0939x; 1.0939x over previous
import jax
import jax.numpy as jnp
from jax import lax
from jax.experimental import pallas as pl
from jax.experimental.pallas import tpu as pltpu
from jax.experimental.pallas import tpu_sc as plsc

N_NODES = 10000
N_EDGES = 320000
D = 128

NC = 2
NS = 16
CHUNK = 80
ROWS_PER_TILE = N_EDGES // (NC * NS * CHUNK)
PIECE = CHUNK
NPIECES = N_NODES // PIECE
PIECES_PER_TILE = -(-NPIECES // NS)
CN = 10240
CPIECE = 640


SUP_C = 2
SUP_E = SUP_C * CHUNK
NSUP = -(-ROWS_PER_TILE // SUP_C)
IDXH = 64
IDXB = ROWS_PER_TILE - IDXH


def _sc_body(src_hbm, idx_hbm, sums_hbm, counts_hbm,
             acc, cnt, idx_v, row0, row1, cstage_v, ones_v,
             ld0, ld1, sc0, sc1, ct0, ct1, idxsem):
    rows = (row0, row1)
    ldsem = (ld0, ld1)
    scsem = (sc0, sc1)
    ctsem = (ct0, ct1)
    c = lax.axis_index("c")
    s = lax.axis_index("s")
    wid = c * NS + s
    base_row = wid * ROWS_PER_TILE

    def super_slice(g):
        return src_hbm.at[pl.ds((base_row + SUP_C * g) * CHUNK, SUP_E)]

    def tail_slice():
        return src_hbm.at[pl.ds((base_row + ROWS_PER_TILE - 1) * CHUNK, CHUNK)]

    idx_cp = pltpu.make_async_copy(
        idx_hbm.at[1, wid, pl.ds(0, IDXH)], idx_v, idxsem)
    idx_cp.start()
    pltpu.async_copy(super_slice(0), rows[0], ldsem[0])

    zf = jnp.zeros((16,), jnp.float32)

    @pl.loop(0, PIECE * D // 16, unroll=8)
    def _(i):
        row1[i // (D // 16), pl.ds((i % (D // 16)) * 16, 16)] = zf

    @pl.loop(0, CHUNK // 16)
    def _(i):
        ones_v[pl.ds(i * 16, 16)] = jnp.ones((16,), jnp.float32)

    zsrc = row1.at[pl.ds(0, PIECE)]
    for k in range(PIECES_PER_TILE):
        p = s + k * NS

        @pl.when(p < NPIECES)
        def _():
            pltpu.async_copy(zsrc, acc.at[pl.ds(p * PIECE, PIECE)], scsem[0])

    for k in range(PIECES_PER_TILE):
        p = s + k * NS

        @pl.when(p < NPIECES)
        def _():
            pltpu.make_async_copy(
                zsrc, acc.at[pl.ds(p * PIECE, PIECE)], scsem[0]).wait()

    @pl.when(s < CN // CPIECE)
    def _():
        @pl.loop(0, CPIECE // 16, unroll=8)
        def _(i):
            cstage_v[pl.ds(i * 16, 16)] = zf
        pltpu.sync_copy(cstage_v, cnt.at[pl.ds(s * CPIECE, CPIECE)])

    plsc.subcore_barrier()

    idx_cp.wait()
    pltpu.async_copy(super_slice(1), rows[1], ldsem[1])

    @pl.loop(0, NSUP, step=2)
    def _(g0):
        for b in range(2):
            g = g0 + b

            @pl.when(g < NSUP)
            def _():
                bn = 1 - b

                @pl.when(g < NSUP - 1)
                def _():
                    pltpu.make_async_copy(
                        super_slice(g), rows[b], ldsem[b]).wait()

                @pl.when(g == NSUP - 1)
                def _():
                    pltpu.make_async_copy(
                        tail_slice(), rows[b].at[pl.ds(0, CHUNK)],
                        ldsem[b]).wait()

                dummy = idx_v.at[0]

                @pl.when(g >= 1)
                def _():
                    for _t in range(SUP_C):
                        pltpu.make_async_copy(
                            rows[bn].at[pl.ds(0, CHUNK)], acc.at[dummy],
                            scsem[bn]).wait()
                        pltpu.make_async_copy(
                            ones_v, cnt.at[dummy], ctsem[bn]).wait()

                @pl.when(g == IDXH // SUP_C)
                def _():
                    pltpu.sync_copy(idx_hbm.at[1, wid, pl.ds(IDXH, IDXB)],
                                    idx_v.at[pl.ds(0, IDXB)])

                for t in range(SUP_C):
                    curc = SUP_C * g + t

                    @pl.when(curc < ROWS_PER_TILE)
                    def _():
                        lrow = jnp.where(curc >= IDXH, curc - IDXH, curc)
                        irow = idx_v.at[lrow]
                        pltpu.async_copy(
                            rows[b].at[pl.ds(t * CHUNK, CHUNK)], acc.at[irow],
                            scsem[b], add=True)
                        pltpu.async_copy(
                            ones_v, cnt.at[irow], ctsem[b], add=True)

                @pl.when((g >= 1) & (g < NSUP - 1))
                def _():
                    @pl.when(g + 1 < NSUP - 1)
                    def _():
                        pltpu.async_copy(
                            super_slice(g + 1), rows[bn], ldsem[bn])

                    @pl.when(g + 1 == NSUP - 1)
                    def _():
                        pltpu.async_copy(
                            tail_slice(), rows[bn].at[pl.ds(0, CHUNK)],
                            ldsem[bn])

    dummy = idx_v.at[0]
    pltpu.make_async_copy(
        rows[0].at[pl.ds(0, CHUNK)], acc.at[dummy], scsem[0]).wait()
    pltpu.make_async_copy(ones_v, cnt.at[dummy], ctsem[0]).wait()

    plsc.subcore_barrier()

    for k in range(PIECES_PER_TILE):
        p = s + k * NS

        @pl.when(p < NPIECES)
        def _():
            off = p * PIECE
            pltpu.async_copy(
                acc.at[pl.ds(off, PIECE)], sums_hbm.at[c, pl.ds(off, PIECE)],
                idxsem)

    @pl.when(s < CN // CPIECE)
    def _():
        pltpu.async_copy(
            cnt.at[pl.ds(s * CPIECE, CPIECE)],
            counts_hbm.at[pl.ds(c * CN + s * CPIECE, CPIECE)], idxsem)

    for k in range(PIECES_PER_TILE):
        p = s + k * NS

        @pl.when(p < NPIECES)
        def _():
            off = p * PIECE
            pltpu.make_async_copy(
                acc.at[pl.ds(off, PIECE)], sums_hbm.at[c, pl.ds(off, PIECE)],
                idxsem).wait()

    @pl.when(s < CN // CPIECE)
    def _():
        pltpu.make_async_copy(
            cnt.at[pl.ds(s * CPIECE, CPIECE)],
            counts_hbm.at[pl.ds(c * CN + s * CPIECE, CPIECE)], idxsem).wait()


_sc_agg = pl.kernel(
    _sc_body,
    out_type=(
        jax.ShapeDtypeStruct((NC, N_NODES, D), jnp.float32),
        jax.ShapeDtypeStruct((NC * CN,), jnp.float32),
    ),
    mesh=plsc.VectorSubcoreMesh(core_axis_name="c", subcore_axis_name="s"),
    scratch_types=[
        pltpu.VMEM_SHARED((N_NODES, D), jnp.float32),
        pltpu.VMEM_SHARED((CN,), jnp.float32),
        pltpu.VMEM((IDXH, CHUNK), jnp.int32),
        pltpu.VMEM((SUP_E, D), jnp.float32),
        pltpu.VMEM((SUP_E, D), jnp.float32),
        pltpu.VMEM((CPIECE,), jnp.float32),
        pltpu.VMEM((CHUNK,), jnp.float32),
    ] + [pltpu.SemaphoreType.DMA] * 7,
)

ROWS_BLK = 2000


def _tc_body(sums_ref, counts_ref, w_ref, b_ref, out_ref):
    total = sums_ref[0] + sums_ref[1]
    cnt = counts_ref[:, 0:1] + counts_ref[:, 1:2]
    mean = total / jnp.maximum(cnt, 1.0)
    out_ref[...] = lax.dot_general(
        mean, w_ref[...], (((1,), (1,)), ((), ())),
        preferred_element_type=jnp.float32,
        precision=lax.Precision.HIGHEST,
    ) + b_ref[...]


_tc_linear = pl.pallas_call(
    _tc_body,
    grid=(N_NODES // ROWS_BLK,),
    in_specs=[
        pl.BlockSpec((NC, ROWS_BLK, D), lambda i: (0, i, 0)),
        pl.BlockSpec((ROWS_BLK, NC), lambda i: (i, 0)),
        pl.BlockSpec((D, D), lambda i: (0, 0)),
        pl.BlockSpec((1, D), lambda i: (0, 0)),
    ],
    out_specs=pl.BlockSpec((ROWS_BLK, D), lambda i: (i, 0)),
    out_shape=jax.ShapeDtypeStruct((N_NODES, D), jnp.float32),
)


def kernel(source_node_representation_with_coefficient, edge_index, feature_dim, W, b):
    src = source_node_representation_with_coefficient
    idx4d = edge_index.astype(jnp.int32).reshape(2, NC * NS, ROWS_PER_TILE, CHUNK)
    sums, counts_flat = _sc_agg(src, idx4d)
    counts_t = counts_flat.reshape(NC, CN)[:, :N_NODES].T
    return _tc_linear(sums, counts_t, W, b.reshape(1, D))


# speedup vs baseline: 1.2339x; 1.1280x over previous
import jax
import jax.numpy as jnp
from jax import lax
from jax.experimental import pallas as pl
from jax.experimental.pallas import tpu as pltpu
from jax.experimental.pallas import tpu_sc as plsc

N_NODES = 10000
N_EDGES = 320000
D = 128

NC = 2
NS = 16
CHUNK = 80
ROWS_PER_TILE = N_EDGES // (NC * NS * CHUNK)
PIECE = CHUNK
NPIECES = N_NODES // PIECE
PIECES_PER_TILE = -(-NPIECES // NS)
CN = 10240
CPIECE = 2048


NB = 3


def _sc_body(src_hbm, idx_hbm, sums_hbm, counts_hbm,
             acc, cnt, idx_v, row0, row1, row2, cstage_v, ones_v,
             ld0, ld1, ld2, sc0, sc1, sc2, ct0, ct1, ct2, idxsem):
    rows = (row0, row1, row2)
    ldsem = (ld0, ld1, ld2)
    scsem = (sc0, sc1, sc2)
    ctsem = (ct0, ct1, ct2)
    zbuf = row2
    c = lax.axis_index("c")
    s = lax.axis_index("s")

    wid = c * NS + s
    base_row = wid * ROWS_PER_TILE

    def src_slice(cur):
        return src_hbm.at[pl.ds((base_row + cur) * CHUNK, CHUNK)]

    idx_cp = pltpu.make_async_copy(idx_hbm.at[1, wid], idx_v, idxsem)
    idx_cp.start()
    pltpu.async_copy(src_slice(0), rows[0], ldsem[0])
    pltpu.async_copy(src_slice(1), rows[1], ldsem[1])

    zf = jnp.zeros((16,), jnp.float32)

    @pl.loop(0, CHUNK * D // 16)
    def _(i):
        zbuf[i // (D // 16), pl.ds((i % (D // 16)) * 16, 16)] = zf

    @pl.loop(0, CHUNK // 16)
    def _(i):
        ones_v[pl.ds(i * 16, 16)] = jnp.ones((16,), jnp.float32)

    for k in range(PIECES_PER_TILE):
        p = s + k * NS

        @pl.when(p < NPIECES)
        def _():
            pltpu.sync_copy(zbuf, acc.at[pl.ds(p * PIECE, PIECE)])

    @pl.when(s < CN // CPIECE)
    def _():
        @pl.loop(0, CPIECE // 16)
        def _(i):
            cstage_v[pl.ds(i * 16, 16)] = zf
        pltpu.sync_copy(cstage_v, cnt.at[pl.ds(s * CPIECE, CPIECE)])

    plsc.subcore_barrier()

    idx_cp.wait()


    @pl.loop(0, ROWS_PER_TILE, step=NB)
    def _(r0):
        for b in range(NB):
            cur = r0 + b

            @pl.when(cur < ROWS_PER_TILE)
            def _():
                pltpu.make_async_copy(src_slice(cur), rows[b], ldsem[b]).wait()
                idx_row = idx_v.at[cur]
                pltpu.async_copy(rows[b], acc.at[idx_row], scsem[b], add=True)
                pltpu.async_copy(ones_v, cnt.at[idx_row], ctsem[b], add=True)
                bn = (b + 2) % NB

                @pl.when(cur >= 1)
                def _():
                    prev_idx = idx_v.at[cur - 1]
                    pltpu.make_async_copy(
                        rows[bn], acc.at[prev_idx], scsem[bn]).wait()
                    pltpu.make_async_copy(
                        ones_v, cnt.at[prev_idx], ctsem[bn]).wait()

                @pl.when(cur + 2 < ROWS_PER_TILE)
                def _():
                    pltpu.async_copy(src_slice(cur + 2), rows[bn], ldsem[bn])

    last = ROWS_PER_TILE - 1
    bl = last % NB
    last_idx = idx_v.at[last]
    pltpu.make_async_copy(rows[bl], acc.at[last_idx], scsem[bl]).wait()
    pltpu.make_async_copy(ones_v, cnt.at[last_idx], ctsem[bl]).wait()

    plsc.subcore_barrier()

    for k in range(PIECES_PER_TILE):
        p = s + k * NS

        @pl.when(p < NPIECES)
        def _():
            off = p * PIECE
            pltpu.async_copy(
                acc.at[pl.ds(off, PIECE)], sums_hbm.at[c, pl.ds(off, PIECE)],
                idxsem)

    @pl.when(s < CN // CPIECE)
    def _():
        pltpu.async_copy(
            cnt.at[pl.ds(s * CPIECE, CPIECE)],
            counts_hbm.at[pl.ds(c * CN + s * CPIECE, CPIECE)], idxsem)

    for k in range(PIECES_PER_TILE):
        p = s + k * NS

        @pl.when(p < NPIECES)
        def _():
            off = p * PIECE
            pltpu.make_async_copy(
                acc.at[pl.ds(off, PIECE)], sums_hbm.at[c, pl.ds(off, PIECE)],
                idxsem).wait()

    @pl.when(s < CN // CPIECE)
    def _():
        pltpu.make_async_copy(
            cnt.at[pl.ds(s * CPIECE, CPIECE)],
            counts_hbm.at[pl.ds(c * CN + s * CPIECE, CPIECE)], idxsem).wait()


_sc_agg = pl.kernel(
    _sc_body,
    out_type=(
        jax.ShapeDtypeStruct((NC, N_NODES, D), jnp.float32),
        jax.ShapeDtypeStruct((NC * CN,), jnp.float32),
    ),
    mesh=plsc.VectorSubcoreMesh(core_axis_name="c", subcore_axis_name="s"),
    scratch_types=[
        pltpu.VMEM_SHARED((N_NODES, D), jnp.float32),
        pltpu.VMEM_SHARED((CN,), jnp.float32),
        pltpu.VMEM((ROWS_PER_TILE, CHUNK), jnp.int32),
        pltpu.VMEM((CHUNK, D), jnp.float32),
        pltpu.VMEM((CHUNK, D), jnp.float32),
        pltpu.VMEM((CHUNK, D), jnp.float32),
        pltpu.VMEM((CPIECE,), jnp.float32),
        pltpu.VMEM((CHUNK,), jnp.float32),
    ] + [pltpu.SemaphoreType.DMA] * 10,
)

ROWS_BLK = 2000


def _tc_body(sums_ref, counts_ref, w_ref, b_ref, out_ref):
    total = sums_ref[0] + sums_ref[1]
    cnt = counts_ref[:, 0:1] + counts_ref[:, 1:2]
    mean = total / jnp.maximum(cnt, 1.0)
    out_ref[...] = lax.dot_general(
        mean, w_ref[...], (((1,), (1,)), ((), ())),
        preferred_element_type=jnp.float32,
        precision=lax.Precision.HIGHEST,
    ) + b_ref[...]


_tc_linear = pl.pallas_call(
    _tc_body,
    grid=(N_NODES // ROWS_BLK,),
    in_specs=[
        pl.BlockSpec((NC, ROWS_BLK, D), lambda i: (0, i, 0)),
        pl.BlockSpec((ROWS_BLK, NC), lambda i: (i, 0)),
        pl.BlockSpec((D, D), lambda i: (0, 0)),
        pl.BlockSpec((1, D), lambda i: (0, 0)),
    ],
    out_specs=pl.BlockSpec((ROWS_BLK, D), lambda i: (i, 0)),
    out_shape=jax.ShapeDtypeStruct((N_NODES, D), jnp.float32),
)


def kernel(source_node_representation_with_coefficient, edge_index, feature_dim, W, b):
    src = source_node_representation_with_coefficient
    idx4d = edge_index.astype(jnp.int32).reshape(2, NC * NS, ROWS_PER_TILE, CHUNK)
    sums, counts_flat = _sc_agg(src, idx4d)
    counts_t = counts_flat.reshape(NC, CN)[:, :N_NODES].T
    return _tc_linear(sums, counts_t, W, b.reshape(1, D))
